# baseline (speedup 1.0000x reference)
import functools

import jax
import jax.numpy as jnp
from jax import lax
from jax.experimental import pallas as pl
from jax.experimental.pallas import tpu as pltpu
from jax.experimental.pallas import tpu_sc as plsc

_B = 16384
_D = 32
_G = 8
_W = 128
_WU = 256
_NC, _NS = 2, 16
_NW = _NC * _NS
_BPW = _B // _NW
_CH = 128
_NCH = _BPW // _CH


@functools.lru_cache(maxsize=None)
def _gather_kernel():
    mesh = plsc.VectorSubcoreMesh(core_axis_name="c", subcore_axis_name="s",
                                  num_cores=_NC, num_subcores=_NS)

    @functools.partial(
        pl.kernel,
        mesh=mesh,
        out_type=(
            jax.ShapeDtypeStruct((_B, _W), jnp.float32),
            jax.ShapeDtypeStruct((_B, _W), jnp.float32),
        ),
        scratch_types=[
            pltpu.VMEM((_NCH, _CH), jnp.int32),
            pltpu.VMEM((_NCH, _CH), jnp.int32),
            pltpu.VMEM((_CH, _W), jnp.float32),
            pltpu.VMEM((_CH, _W), jnp.float32),
            pltpu.SemaphoreType.DMA,
            pltpu.SemaphoreType.DMA,
        ],
        compiler_params=pltpu.CompilerParams(use_tc_tiling_on_sc=True),
    )
    def _gather(ugid_hbm, igid_hbm, gu_hbm, gi_hbm,
                uout_hbm, iout_hbm,
                uidx_v, iidx_v, urows_v, irows_v, usem, isem):
        wid = lax.axis_index("s") * _NC + lax.axis_index("c")
        base = wid * _BPW
        for j in range(_NCH):
            pltpu.sync_copy(ugid_hbm.at[pl.ds(base + j * _CH, _CH)],
                            uidx_v.at[j])
            pltpu.sync_copy(igid_hbm.at[pl.ds(base + j * _CH, _CH)],
                            iidx_v.at[j])
        for j in range(_NCH):
            gu = pltpu.async_copy(gu_hbm.at[uidx_v.at[j]], urows_v, usem)
            gi = pltpu.async_copy(gi_hbm.at[iidx_v.at[j]], irows_v, isem)
            gu.wait()
            pltpu.sync_copy(urows_v, uout_hbm.at[pl.ds(base + j * _CH, _CH)])
            gi.wait()
            pltpu.sync_copy(irows_v, iout_hbm.at[pl.ds(base + j * _CH, _CH)])

    return _gather


_BM = 2048


def _mlp_body(u_ref, v_ref, usel_ref, vsel_ref, w1u_ref, w1v_ref, b1_ref,
              w2_ref, b2_ref, w3_ref, b3_ref, o_ref):
    sub = lax.broadcasted_iota(jnp.int32, (_BM, _WU), 1) // _D
    xu = jnp.where(sub == usel_ref[...], u_ref[...], 0.0)
    xv = jnp.where(sub == vsel_ref[...], v_ref[...], 0.0)
    x1 = (jnp.dot(xu, w1u_ref[...], preferred_element_type=jnp.float32)
          + jnp.dot(xv, w1v_ref[...], preferred_element_type=jnp.float32)
          + b1_ref[...])
    h1 = jnp.maximum(x1, 0.0)
    h2 = jnp.maximum(
        jnp.dot(h1, w2_ref[...], preferred_element_type=jnp.float32)
        + b2_ref[...], 0.0)
    o_ref[...] = (jnp.dot(h2, w3_ref[...], preferred_element_type=jnp.float32)
                  + b3_ref[...])


def _mlp(u256, i256, usel, isel, W1u8, W1i8, b1, W2, b2, W3, b3):
    return pl.pallas_call(
        _mlp_body,
        grid=(_B // _BM,),
        in_specs=[
            pl.BlockSpec((_BM, _WU), lambda m: (m, 0)),
            pl.BlockSpec((_BM, _WU), lambda m: (m, 0)),
            pl.BlockSpec((_BM, 1), lambda m: (m, 0)),
            pl.BlockSpec((_BM, 1), lambda m: (m, 0)),
            pl.BlockSpec((_WU, 64), lambda m: (0, 0)),
            pl.BlockSpec((_WU, 64), lambda m: (0, 0)),
            pl.BlockSpec((1, 64), lambda m: (0, 0)),
            pl.BlockSpec((64, 32), lambda m: (0, 0)),
            pl.BlockSpec((1, 32), lambda m: (0, 0)),
            pl.BlockSpec((32, 1), lambda m: (0, 0)),
            pl.BlockSpec((1, 1), lambda m: (0, 0)),
        ],
        out_specs=pl.BlockSpec((_BM, 1), lambda m: (m, 0)),
        out_shape=jax.ShapeDtypeStruct((_B, 1), jnp.float32),
    )(u256, i256, usel, isel, W1u8, W1i8, b1.reshape(1, 64),
      W2, b2.reshape(1, 32), W3, b3.reshape(1, 1))


def _pack(table):
    t16 = table.astype(jnp.bfloat16)
    return lax.bitcast_convert_type(
        t16.reshape(-1, 16, 2), jnp.float32).reshape(-1, _W)


def _unpack(x):
    return lax.bitcast_convert_type(
        x, jnp.bfloat16).reshape(_B, _WU).astype(jnp.float32)


def kernel(inputs, user_table, item_table, W1, b1, W2, b2, W3, b3):
    idx = inputs.astype(jnp.int32)
    ugid = idx[:, 0] >> 3
    igid = idx[:, 1] >> 3
    usel = (idx[:, 0] & 7).reshape(_B, 1)
    isel = (idx[:, 1] & 7).reshape(_B, 1)
    u128p, i128p = _gather_kernel()(ugid, igid, _pack(user_table),
                                    _pack(item_table))
    W1u8 = jnp.tile(W1[:_D, :], (_G, 1))
    W1i8 = jnp.tile(W1[_D:, :], (_G, 1))
    return _mlp(_unpack(u128p), _unpack(i128p), usel, isel, W1u8, W1i8,
                b1, W2, b2, W3, b3)


# speedup vs baseline: 3.4218x; 3.4218x over previous
import functools

import jax
import jax.numpy as jnp
from jax import lax
from jax.experimental import pallas as pl
from jax.experimental.pallas import tpu as pltpu
from jax.experimental.pallas import tpu_sc as plsc

_B = 16384
_D = 32
_NC, _NS = 2, 16
_NW = _NC * _NS
_BPW = _B // _NW
_CH = 32
_NCHK = _BPW // _CH


@functools.lru_cache(maxsize=None)
def _gather_pairs_kernel():
    mesh = plsc.VectorSubcoreMesh(core_axis_name="c", subcore_axis_name="s",
                                  num_cores=_NC, num_subcores=_NS)

    @functools.partial(
        pl.kernel,
        mesh=mesh,
        out_type=(
            jax.ShapeDtypeStruct((_B, _D), jnp.float32),
            jax.ShapeDtypeStruct((_B, _D), jnp.float32),
        ),
        scratch_types=[
            pltpu.VMEM((_BPW,), jnp.int32),
            pltpu.VMEM((_BPW,), jnp.int32),
            pltpu.VMEM((_CH, _D), jnp.float32),
            pltpu.VMEM((_CH, _D), jnp.float32),
            pltpu.SemaphoreType.DMA,
            pltpu.SemaphoreType.DMA,
        ],
        compiler_params=pltpu.CompilerParams(use_tc_tiling_on_sc=True),
    )
    def _gather_pairs(uidx_hbm, iidx_hbm, utab_hbm, itab_hbm,
                      uout_hbm, iout_hbm,
                      uidx_v, iidx_v, uchunk, ichunk, usem, isem):
        wid = lax.axis_index("s") * _NC + lax.axis_index("c")
        base = wid * _BPW
        pltpu.sync_copy(uidx_hbm.at[pl.ds(base, _BPW)], uidx_v)
        pltpu.sync_copy(iidx_hbm.at[pl.ds(base, _BPW)], iidx_v)

        def body(j, carry):
            uvec = uidx_v[pl.ds(j * _CH, _CH)]
            ivec = iidx_v[pl.ds(j * _CH, _CH)]
            hs = []
            for k in range(_CH):
                hs.append(pltpu.async_copy(
                    utab_hbm.at[pl.ds(uvec[k], 1)],
                    uchunk.at[pl.ds(k, 1)], usem))
                hs.append(pltpu.async_copy(
                    itab_hbm.at[pl.ds(ivec[k], 1)],
                    ichunk.at[pl.ds(k, 1)], isem))
            for h in hs:
                h.wait()
            pltpu.sync_copy(uchunk, uout_hbm.at[pl.ds(base + j * _CH, _CH)])
            pltpu.sync_copy(ichunk, iout_hbm.at[pl.ds(base + j * _CH, _CH)])
            return carry

        lax.fori_loop(0, _NCHK, body, 0)

    return _gather_pairs


_BM = 2048


def _mlp_body(u_ref, v_ref, w1_ref, b1_ref, w2_ref, b2_ref, w3_ref, b3_ref,
              o_ref):
    x1 = (jnp.dot(u_ref[...], w1_ref[0:_D, :],
                  preferred_element_type=jnp.float32)
          + jnp.dot(v_ref[...], w1_ref[_D:2 * _D, :],
                    preferred_element_type=jnp.float32)
          + b1_ref[...])
    h1 = jnp.maximum(x1, 0.0)
    h2 = jnp.maximum(
        jnp.dot(h1, w2_ref[...], preferred_element_type=jnp.float32)
        + b2_ref[...], 0.0)
    o_ref[...] = (jnp.dot(h2, w3_ref[...], preferred_element_type=jnp.float32)
                  + b3_ref[...])


def _mlp(u_vec, i_vec, W1, b1, W2, b2, W3, b3):
    return pl.pallas_call(
        _mlp_body,
        grid=(_B // _BM,),
        in_specs=[
            pl.BlockSpec((_BM, _D), lambda m: (m, 0)),
            pl.BlockSpec((_BM, _D), lambda m: (m, 0)),
            pl.BlockSpec((2 * _D, 64), lambda m: (0, 0)),
            pl.BlockSpec((1, 64), lambda m: (0, 0)),
            pl.BlockSpec((64, 32), lambda m: (0, 0)),
            pl.BlockSpec((1, 32), lambda m: (0, 0)),
            pl.BlockSpec((32, 1), lambda m: (0, 0)),
            pl.BlockSpec((1, 1), lambda m: (0, 0)),
        ],
        out_specs=pl.BlockSpec((_BM, 1), lambda m: (m, 0)),
        out_shape=jax.ShapeDtypeStruct((_B, 1), jnp.float32),
    )(u_vec, i_vec, W1, b1.reshape(1, 64), W2, b2.reshape(1, 32),
      W3, b3.reshape(1, 1))


def kernel(inputs, user_table, item_table, W1, b1, W2, b2, W3, b3):
    idx = inputs.astype(jnp.int32)
    uidx = idx[:, 0]
    iidx = idx[:, 1]
    u_vec, i_vec = _gather_pairs_kernel()(uidx, iidx, user_table, item_table)
    return _mlp(u_vec, i_vec, W1, b1, W2, b2, W3, b3)
